# baseline (speedup 1.0000x reference)
import functools

import jax
import jax.numpy as jnp
from jax.experimental import pallas as pl
from jax.experimental.pallas import tpu as pltpu

_BLOCK_K = 256


def _router_kernel(x_ref, w_ref, o_ref):
    k = pl.program_id(0)
    acc = jax.lax.dot_general(
        x_ref[...],
        w_ref[...],
        (((1,), (1,)), ((), ())),
        preferred_element_type=jnp.float32,
    )

    @pl.when(k == 0)
    def _():
        o_ref[...] = acc

    @pl.when(k > 0)
    def _():
        o_ref[...] += acc


@functools.partial(jax.jit, static_argnames=())
def kernel(x, gate_w):
    b, t, d = x.shape
    e = gate_w.shape[0]
    m = b * t
    x2 = x.reshape(m, d)

    out = pl.pallas_call(
        _router_kernel,
        grid=(d // _BLOCK_K,),
        in_specs=[
            pl.BlockSpec((m, _BLOCK_K), lambda k: (0, k)),
            pl.BlockSpec((e, _BLOCK_K), lambda k: (0, k)),
        ],
        out_specs=pl.BlockSpec((m, e), lambda k: (0, 0)),
        out_shape=jax.ShapeDtypeStruct((m, e), jnp.float32),
        compiler_params=pltpu.CompilerParams(
            dimension_semantics=("arbitrary",),
        ),
    )(x2, gate_w)
    return out.reshape(b, t, e)


# speedup vs baseline: 1.0254x; 1.0254x over previous
import functools

import jax
import jax.numpy as jnp
from jax.experimental import pallas as pl
from jax.experimental.pallas import tpu as pltpu

_CHUNK = 512
_NBUF = 4


def _router_kernel(x_hbm, w_ref, o_hbm, *scratch):
    xbufs = scratch[:_NBUF]
    obuf = scratch[_NBUF]
    in_sems = scratch[_NBUF + 1]
    out_sems = scratch[_NBUF + 2]
    m = x_hbm.shape[0]
    n_chunks = m // _CHUNK
    tail_rows = (n_chunks - 1) * _CHUNK

    def in_copy(i):
        slot = i % _NBUF
        return pltpu.make_async_copy(
            x_hbm.at[pl.ds(i * _CHUNK, _CHUNK), :],
            xbufs[slot],
            in_sems.at[slot],
        )

    bulk_out = pltpu.make_async_copy(
        obuf.at[pl.ds(0, tail_rows), :],
        o_hbm.at[pl.ds(0, tail_rows), :],
        out_sems.at[0],
    )
    tail_out = pltpu.make_async_copy(
        obuf.at[pl.ds(tail_rows, _CHUNK), :],
        o_hbm.at[pl.ds(tail_rows, _CHUNK), :],
        out_sems.at[1],
    )

    wb = w_ref[...].astype(jnp.bfloat16)

    for s in range(min(_NBUF, n_chunks)):
        in_copy(s).start()

    for i in range(n_chunks):
        in_copy(i).wait()
        slot = i % _NBUF
        obuf[pl.ds(i * _CHUNK, _CHUNK), :] = jax.lax.dot_general(
            xbufs[slot][...].astype(jnp.bfloat16),
            wb,
            (((1,), (1,)), ((), ())),
            preferred_element_type=jnp.float32,
        )
        if i + _NBUF < n_chunks:
            in_copy(i + _NBUF).start()
        if i == n_chunks - 2:
            bulk_out.start()

    tail_out.start()
    bulk_out.wait()
    tail_out.wait()


@functools.partial(jax.jit, static_argnames=())
def kernel(x, gate_w):
    b, t, d = x.shape
    e = gate_w.shape[0]
    m = b * t
    x2 = x.reshape(m, d)

    out = pl.pallas_call(
        _router_kernel,
        in_specs=[
            pl.BlockSpec(memory_space=pl.ANY),
            pl.BlockSpec(memory_space=pltpu.VMEM),
        ],
        out_specs=pl.BlockSpec(memory_space=pl.ANY),
        out_shape=jax.ShapeDtypeStruct((m, e), jnp.float32),
        scratch_shapes=(
            [pltpu.VMEM((_CHUNK, d), jnp.float32) for _ in range(_NBUF)]
            + [pltpu.VMEM((m, e), jnp.float32),
               pltpu.SemaphoreType.DMA((_NBUF,)),
               pltpu.SemaphoreType.DMA((2,))]
        ),
    )(x2, gate_w)
    return out.reshape(b, t, e)
